# baseline (speedup 1.0000x reference)
import jax
import jax.numpy as jnp
from jax import lax
from jax.experimental import pallas as pl
from jax.experimental.pallas import tpu as pltpu
from jax.experimental.pallas import tpu_sc as plsc

N, D, B = 10000, 256, 64
NC, NS, L = 2, 16, 16
NW = NC * NS
GR = D // L
CHUNK = 80
NCHUNK = N // CHUNK
ITERS = (NCHUNK + NW - 1) // NW


def _sc_body(x_hbm, b_hbm, sums_out, cnt_out,
             xb0, xb1, ib0, ib1, acc_v, cnt_v, xs0, xs1, is0, is1):
    cid = lax.axis_index("c")
    sid = lax.axis_index("s")
    wid = sid * NC + cid

    zero = jnp.zeros((L,), jnp.float32)
    one = jnp.ones((L,), jnp.float32)
    blk = jnp.full((L,), float(L), jnp.float32)

    xbufs, ibufs = (xb0, xb1), (ib0, ib1)
    xsems, isems = (xs0, xs1), (is0, is1)

    def start(j):
        c = wid + NW * j

        @pl.when(c < NCHUNK)
        def _(j=j, c=c):
            base = c * CHUNK
            pltpu.async_copy(x_hbm.at[pl.ds(base, CHUNK)], xbufs[j % 2], xsems[j % 2])
            pltpu.async_copy(b_hbm.at[pl.ds(base, CHUNK)], ibufs[j % 2], isems[j % 2])

    def process(xbuf, idxbuf):
        def block(k, carry):
            segs = idxbuf[pl.ds(k * L, L)]
            s_first = segs[0]
            s_last = segs[L - 1]

            @pl.when(s_first == s_last)
            def _fast():
                for g in range(GR):
                    sl = pl.ds(g * L, L)
                    vals = [xbuf[k * L + r, sl] for r in range(L)]
                    while len(vals) > 1:
                        vals = [vals[i] + vals[i + 1] for i in range(0, len(vals), 2)]
                    acc_v[s_first, sl] += vals[0]
                cnt_v[s_first, :] += blk

            @pl.when(s_first != s_last)
            def _slow():
                for r in range(L):
                    s = segs[r]
                    xs = [xbuf[k * L + r, pl.ds(g * L, L)] for g in range(GR)]
                    for g in range(GR):
                        acc_v[s, pl.ds(g * L, L)] += xs[g]
                    cnt_v[s, :] += one

            return carry

        lax.fori_loop(0, CHUNK // L, block, 0)

    def zrow(r, c):
        for g in range(GR):
            acc_v[r, pl.ds(g * L, L)] = zero
        cnt_v[r, :] = zero
        return c

    lax.fori_loop(0, B, zrow, 0)

    for j in range(ITERS):
        c = wid + NW * j

        @pl.when(c < NCHUNK)
        def _(j=j, c=c):
            base = c * CHUNK
            pltpu.sync_copy(x_hbm.at[pl.ds(base, CHUNK)], xbufs[0])
            pltpu.sync_copy(b_hbm.at[pl.ds(base, CHUNK)], ibufs[0])
            process(xbufs[0], ibufs[0])

    pltpu.sync_copy(acc_v, sums_out.at[wid])
    pltpu.sync_copy(cnt_v, cnt_out.at[wid])


_sc_segsum = pl.kernel(
    _sc_body,
    out_type=[
        jax.ShapeDtypeStruct((NW, B, D), jnp.float32),
        jax.ShapeDtypeStruct((NW, B, L), jnp.float32),
    ],
    mesh=plsc.VectorSubcoreMesh(
        core_axis_name="c", subcore_axis_name="s", num_cores=NC, num_subcores=NS
    ),
    scratch_types=[
        pltpu.VMEM((CHUNK, D), jnp.float32),
        pltpu.VMEM((CHUNK, D), jnp.float32),
        pltpu.VMEM((CHUNK,), jnp.int32),
        pltpu.VMEM((CHUNK,), jnp.int32),
        pltpu.VMEM((B, D), jnp.float32),
        pltpu.VMEM((B, L), jnp.float32),
        pltpu.SemaphoreType.DMA,
        pltpu.SemaphoreType.DMA,
        pltpu.SemaphoreType.DMA,
        pltpu.SemaphoreType.DMA,
    ],
)


def _mlp_body(ps_ref, pc_ref, u_ref, w1_ref, b1_ref, w2_ref, b2_ref, out_ref):
    sums = jnp.sum(ps_ref[...], axis=0)
    cnt = jnp.sum(pc_ref[...], axis=0)
    mean = sums / jnp.clip(cnt[:, :1], 1.0, None)
    cat = jnp.concatenate([u_ref[...], mean], axis=1)
    h = (jnp.dot(cat, w1_ref[...], preferred_element_type=jnp.float32)
         + b1_ref[...][None, :])
    h = jnp.maximum(h, 0.0)
    out_ref[...] = (
        jnp.dot(h, w2_ref[...], preferred_element_type=jnp.float32)
        + b2_ref[...][None, :]
    )


def _tc_mlp(ps, pc, u, w1, b1, w2, b2):
    return pl.pallas_call(
        _mlp_body,
        out_shape=jax.ShapeDtypeStruct((B, D), jnp.float32),
    )(ps, pc, u, w1, b1, w2, b2)


def kernel(x, edge_index, edge_attr, u, batch, W1, b1, W2, b2):
    del edge_index, edge_attr
    bi = batch.astype(jnp.int32)
    sums_p, cnt_p = _sc_segsum(x, bi)
    return _tc_mlp(sums_p, cnt_p, u, W1, b1, W2, b2)


# speedup vs baseline: 1.1197x; 1.1197x over previous
import jax
import jax.numpy as jnp
from jax import lax
from jax.experimental import pallas as pl
from jax.experimental.pallas import tpu as pltpu
from jax.experimental.pallas import tpu_sc as plsc

N, D, B = 10000, 256, 64
NC, NS, L = 2, 16, 16
NW = NC * NS
GR = D // L
CHUNK = 80
NCHUNK = N // CHUNK
ITERS = (NCHUNK + NW - 1) // NW


def _sc_body(x_hbm, b_hbm, sums_out, cnt_out,
             xb0, xb1, ib0, ib1, acc_v, cnt_v, xs0, xs1, is0, is1):
    cid = lax.axis_index("c")
    sid = lax.axis_index("s")
    wid = sid * NC + cid

    zero = jnp.zeros((L,), jnp.float32)
    one = jnp.ones((L,), jnp.float32)
    blk = jnp.full((L,), float(L), jnp.float32)

    xbufs, ibufs = (xb0, xb1), (ib0, ib1)
    xsems, isems = (xs0, xs1), (is0, is1)

    def start(j):
        c = wid + NW * j

        @pl.when(c < NCHUNK)
        def _(j=j, c=c):
            base = c * CHUNK
            pltpu.async_copy(x_hbm.at[pl.ds(base, CHUNK)], xbufs[j % 2], xsems[j % 2])
            pltpu.async_copy(b_hbm.at[pl.ds(base, CHUNK)], ibufs[j % 2], isems[j % 2])

    def process(xbuf, idxbuf):
        def block(k, carry):
            segs = idxbuf[pl.ds(k * L, L)]
            s_first = segs[0]
            s_last = segs[L - 1]

            @pl.when(s_first == s_last)
            def _fast():
                for g in range(GR):
                    sl = pl.ds(g * L, L)
                    vals = [xbuf[k * L + r, sl] for r in range(L)]
                    while len(vals) > 1:
                        vals = [vals[i] + vals[i + 1] for i in range(0, len(vals), 2)]
                    acc_v[s_first, sl] += vals[0]
                cnt_v[s_first, :] += blk

            @pl.when(s_first != s_last)
            def _slow():
                for r in range(L):
                    s = segs[r]
                    xs = [xbuf[k * L + r, pl.ds(g * L, L)] for g in range(GR)]
                    for g in range(GR):
                        acc_v[s, pl.ds(g * L, L)] += xs[g]
                    cnt_v[s, :] += one

            return carry

        lax.fori_loop(0, CHUNK // L, block, 0)

    start(0)

    def zrow(r, c):
        for g in range(GR):
            acc_v[r, pl.ds(g * L, L)] = zero
        cnt_v[r, :] = zero
        return c

    lax.fori_loop(0, B, zrow, 0)

    for j in range(ITERS):
        if j + 1 < ITERS:
            start(j + 1)
        c = wid + NW * j

        @pl.when(c < NCHUNK)
        def _(j=j, c=c):
            base = c * CHUNK
            pltpu.make_async_copy(
                x_hbm.at[pl.ds(base, CHUNK)], xbufs[j % 2], xsems[j % 2]).wait()
            pltpu.make_async_copy(
                b_hbm.at[pl.ds(base, CHUNK)], ibufs[j % 2], isems[j % 2]).wait()
            process(xbufs[j % 2], ibufs[j % 2])

    pltpu.sync_copy(acc_v, sums_out.at[wid])
    pltpu.sync_copy(cnt_v, cnt_out.at[wid])


_sc_segsum = pl.kernel(
    _sc_body,
    out_type=[
        jax.ShapeDtypeStruct((NW, B, D), jnp.float32),
        jax.ShapeDtypeStruct((NW, B, L), jnp.float32),
    ],
    mesh=plsc.VectorSubcoreMesh(
        core_axis_name="c", subcore_axis_name="s", num_cores=NC, num_subcores=NS
    ),
    scratch_types=[
        pltpu.VMEM((CHUNK, D), jnp.float32),
        pltpu.VMEM((CHUNK, D), jnp.float32),
        pltpu.VMEM((CHUNK,), jnp.int32),
        pltpu.VMEM((CHUNK,), jnp.int32),
        pltpu.VMEM((B, D), jnp.float32),
        pltpu.VMEM((B, L), jnp.float32),
        pltpu.SemaphoreType.DMA,
        pltpu.SemaphoreType.DMA,
        pltpu.SemaphoreType.DMA,
        pltpu.SemaphoreType.DMA,
    ],
)


def _mlp_body(ps_ref, pc_ref, u_ref, w1_ref, b1_ref, w2_ref, b2_ref, out_ref):
    sums = jnp.sum(ps_ref[...], axis=0)
    cnt = jnp.sum(pc_ref[...], axis=0)
    mean = sums / jnp.clip(cnt[:, :1], 1.0, None)
    cat = jnp.concatenate([u_ref[...], mean], axis=1)
    h = (jnp.dot(cat, w1_ref[...], preferred_element_type=jnp.float32)
         + b1_ref[...][None, :])
    h = jnp.maximum(h, 0.0)
    out_ref[...] = (
        jnp.dot(h, w2_ref[...], preferred_element_type=jnp.float32)
        + b2_ref[...][None, :]
    )


def _tc_mlp(ps, pc, u, w1, b1, w2, b2):
    return pl.pallas_call(
        _mlp_body,
        out_shape=jax.ShapeDtypeStruct((B, D), jnp.float32),
    )(ps, pc, u, w1, b1, w2, b2)


def kernel(x, edge_index, edge_attr, u, batch, W1, b1, W2, b2):
    del edge_index, edge_attr
    bi = batch.astype(jnp.int32)
    sums_p, cnt_p = _sc_segsum(x, bi)
    return _tc_mlp(sums_p, cnt_p, u, W1, b1, W2, b2)
